# baseline (speedup 1.0000x reference)
import functools

import jax
import jax.numpy as jnp
from jax import lax
from jax.experimental import pallas as pl
from jax.experimental.pallas import tpu as pltpu
from jax.experimental.pallas import tpu_sc as plsc

_NC = 2
_NS = 16
_NW = _NC * _NS
_NBUF = 4


def _make_sc_kernel(batch, hist, dim):
    assert dim == 64 and batch % (_NW * _NBUF) == 0 and hist % 8 == 0
    bpw = batch // _NW
    ch0 = min(hist, 128)
    ch1 = hist - ch0
    thresh = jnp.uint32(hist // 2)
    mesh = plsc.VectorSubcoreMesh(core_axis_name="c", subcore_axis_name="s")

    @functools.partial(
        pl.kernel,
        out_type=jax.ShapeDtypeStruct((batch, dim), jnp.float32),
        mesh=mesh,
        scratch_types=[
            pltpu.VMEM((bpw, hist), jnp.int32),
            pltpu.VMEM((_NBUF, hist, 16), jnp.uint32),
            pltpu.VMEM((bpw, dim), jnp.float32),
            [pltpu.SemaphoreType.DMA] * _NBUF,
        ],
        compiler_params=pltpu.CompilerParams(
            use_tc_tiling_on_sc=False, needs_layout_passes=False),
    )
    def sc_kernel(idx_hbm, packed_hbm, out_hbm, idx_v, buf_v, out_v, sems):
        wid = lax.axis_index("s") * _NC + lax.axis_index("c")
        pltpu.sync_copy(idx_hbm.at[pl.ds(wid * bpw, bpw)], idx_v)

        def issue(b, s):
            pltpu.async_copy(
                packed_hbm.at[idx_v.at[b, pl.ds(0, ch0)]],
                buf_v.at[s, pl.ds(0, ch0)], sems[s])
            pltpu.async_copy(
                packed_hbm.at[idx_v.at[b, pl.ds(ch0, ch1)]],
                buf_v.at[s, pl.ds(ch0, ch1)], sems[s])

        def wait(b, s):
            pltpu.make_async_copy(
                packed_hbm.at[idx_v.at[b, pl.ds(0, ch0)]],
                buf_v.at[s, pl.ds(0, ch0)], sems[s]).wait()
            pltpu.make_async_copy(
                packed_hbm.at[idx_v.at[b, pl.ds(ch0, ch1)]],
                buf_v.at[s, pl.ds(ch0, ch1)], sems[s]).wait()

        def process(b, s):
            zero = jnp.zeros((16,), jnp.uint32)
            u = 8

            def red(j, acc):
                out = []
                for i in range(u):
                    out.append(acc[i] + buf_v[s, j * u + i, :])
                return tuple(out)

            accs = lax.fori_loop(0, hist // u, red, (zero,) * u)
            a = accs[0]
            for t in accs[1:]:
                a = a + t

            m = jnp.uint32(0xFF)
            one = jnp.float32(1.0)
            neg = jnp.float32(-1.0)
            for j in range(4):
                c = (a >> (8 * j)) & m
                out_v[b, pl.ds(16 * j, 16)] = jnp.where(c > thresh, one, neg)

        for s in range(_NBUF):
            issue(s, s)

        def outer(i, _):
            b0 = i * _NBUF
            for s in range(_NBUF):
                b = b0 + s
                wait(b, s)
                process(b, s)
                nb = b + _NBUF

                @pl.when(nb < bpw)
                def _():
                    issue(nb, s)
            return 0

        lax.fori_loop(0, bpw // _NBUF, outer, 0)
        pltpu.sync_copy(out_v, out_hbm.at[pl.ds(wid * bpw, bpw)])

    return sc_kernel


def kernel(x, table):
    batch, hist = x.shape
    size, dim = table.shape
    b = (table > 0).astype(jnp.uint32)
    packed = (b[:, 0:16] | (b[:, 16:32] << 8) | (b[:, 32:48] << 16)
              | (b[:, 48:64] << 24))
    idx = x.astype(jnp.int32)
    sc = _make_sc_kernel(batch, hist, dim)
    return sc(idx, packed)


# speedup vs baseline: 1.4012x; 1.4012x over previous
import functools

import jax
import jax.numpy as jnp
from jax import lax
from jax.experimental import pallas as pl
from jax.experimental.pallas import tpu as pltpu
from jax.experimental.pallas import tpu_sc as plsc

_NC = 2
_NS = 16
_NW = _NC * _NS
_NBUF = 4


def _make_sc_kernel(batch, hist, dim):
    assert dim == 64 and batch % (_NW * _NBUF) == 0 and hist % 8 == 0
    bpw = batch // _NW
    ch0 = min(hist, 128)
    ch1 = hist - ch0
    thresh = jnp.uint32(hist // 2)
    mesh = plsc.VectorSubcoreMesh(core_axis_name="c", subcore_axis_name="s")

    @functools.partial(
        pl.kernel,
        out_type=jax.ShapeDtypeStruct((batch, dim), jnp.float32),
        mesh=mesh,
        scratch_types=[
            pltpu.VMEM((bpw, hist), jnp.int32),
            pltpu.VMEM((_NBUF, hist, 64), jnp.uint8),
            pltpu.VMEM((bpw, dim), jnp.float32),
            [pltpu.SemaphoreType.DMA] * _NBUF,
        ],
        compiler_params=pltpu.CompilerParams(
            use_tc_tiling_on_sc=False, needs_layout_passes=False),
    )
    def sc_kernel(idx_hbm, packed_hbm, out_hbm, idx_v, buf_v, out_v, sems):
        wid = lax.axis_index("s") * _NC + lax.axis_index("c")
        pltpu.sync_copy(idx_hbm.at[pl.ds(wid * bpw, bpw)], idx_v)

        def issue(b, s):
            pltpu.async_copy(
                packed_hbm.at[idx_v.at[b, pl.ds(0, ch0)]],
                buf_v.at[s, pl.ds(0, ch0)], sems[s])
            pltpu.async_copy(
                packed_hbm.at[idx_v.at[b, pl.ds(ch0, ch1)]],
                buf_v.at[s, pl.ds(ch0, ch1)], sems[s])

        def wait(b, s):
            pltpu.make_async_copy(
                packed_hbm.at[idx_v.at[b, pl.ds(0, ch0)]],
                buf_v.at[s, pl.ds(0, ch0)], sems[s]).wait()
            pltpu.make_async_copy(
                packed_hbm.at[idx_v.at[b, pl.ds(ch0, ch1)]],
                buf_v.at[s, pl.ds(ch0, ch1)], sems[s]).wait()

        def process(b, s):
            zero = jnp.zeros((16,), jnp.uint32)
            u = 8

            def red(j, acc):
                out = []
                for i in range(u):
                    r = plsc.bitcast(buf_v[s, j * u + i, :], jnp.uint32)
                    out.append(acc[i] + r)
                return tuple(out)

            accs = lax.fori_loop(0, hist // u, red, (zero,) * u)
            a = accs[0]
            for t in accs[1:]:
                a = a + t

            m = jnp.uint32(0xFF)
            lane4 = 4 * lax.iota(jnp.int32, 16)
            b_vec = jnp.full((16,), b, jnp.int32)
            one = jnp.float32(1.0)
            neg = jnp.float32(-1.0)
            for j in range(4):
                c = (a >> (8 * j)) & m
                plsc.store_scatter(
                    out_v, [b_vec, lane4 + j], jnp.where(c > thresh, one, neg))

        for s in range(_NBUF):
            issue(s, s)

        def outer(i, _):
            b0 = i * _NBUF
            for s in range(_NBUF):
                b = b0 + s
                wait(b, s)
                process(b, s)
                nb = b + _NBUF

                @pl.when(nb < bpw)
                def _():
                    issue(nb, s)
            return 0

        lax.fori_loop(0, bpw // _NBUF, outer, 0)
        pltpu.sync_copy(out_v, out_hbm.at[pl.ds(wid * bpw, bpw)])

    return sc_kernel


def kernel(x, table):
    batch, hist = x.shape
    size, dim = table.shape
    bits = (table > 0).astype(jnp.uint8)
    idx = x.astype(jnp.int32)
    sc = _make_sc_kernel(batch, hist, dim)
    return sc(idx, bits)
